# Initial kernel scaffold; baseline (speedup 1.0000x reference)
import functools

import jax
import jax.numpy as jnp
from jax import lax
from jax.experimental import pallas as pl
from jax.experimental.pallas import tpu as pltpu
import jax.experimental.pallas.tpu_sc as plsc

N = 10000
NPAD = 10240
E = 160000
G = 100
EPS = 1e-5
C = 40
NCH = E // C
NC, NS = 2, 16
NW = NC * NS
EB = 2000
NEB = E // EB
NB = 1250
NNB = N // NB

_SC_MESH = dict(core_axis_name="c", subcore_axis_name="s")


@functools.partial(jax.jit, static_argnames=("d",))
def _sc_gather_add(p, q, dstc, srcc, *, d):
    nf = d // 16
    cpt = NCH // NW

    @functools.partial(
        pl.kernel,
        out_type=(
            jax.ShapeDtypeStruct((E, d), jnp.float32),
            jax.ShapeDtypeStruct((NW, 2, d), jnp.float32),
        ),
        mesh=plsc.VectorSubcoreMesh(**_SC_MESH),
        scratch_types=[
            pltpu.VMEM((C,), jnp.int32),
            pltpu.VMEM((C,), jnp.int32),
            pltpu.VMEM((C, d), jnp.float32),
            pltpu.VMEM((C, d), jnp.float32),
            pltpu.VMEM((2, d), jnp.float32),
            pltpu.SemaphoreType.DMA,
            pltpu.SemaphoreType.DMA,
        ],
    )
    def k(p_hbm, q_hbm, dst_hbm, src_hbm, y_hbm, st_hbm,
          dbuf, sbuf, pbuf, qbuf, stbuf, sem1, sem2):
        wid = lax.axis_index("s") * NC + lax.axis_index("c")
        base = wid * cpt
        zero = jnp.zeros((16,), jnp.float32)
        init = tuple(zero for _ in range(2 * nf))

        def chunk_body(i, carry):
            j = base + i
            pltpu.sync_copy(dst_hbm.at[j], dbuf)
            pltpu.sync_copy(src_hbm.at[j], sbuf)
            cp1 = pltpu.async_copy(p_hbm.at[dbuf], pbuf, sem1)
            cp2 = pltpu.async_copy(q_hbm.at[sbuf], qbuf, sem2)
            cp1.wait()
            cp2.wait()

            def row_body(r, rc):
                acc = list(rc)
                for f in range(nf):
                    y = pbuf[r, pl.ds(f * 16, 16)] + qbuf[r, pl.ds(f * 16, 16)]
                    pbuf[r, pl.ds(f * 16, 16)] = y
                    acc[f] = acc[f] + y
                    acc[nf + f] = acc[nf + f] + y * y
                return tuple(acc)

            carry = lax.fori_loop(0, C, row_body, carry)
            pltpu.sync_copy(pbuf, y_hbm.at[pl.ds(j * C, C), :])
            return carry

        carry = lax.fori_loop(0, cpt, chunk_body, init)
        for f in range(nf):
            stbuf[0, pl.ds(f * 16, 16)] = carry[f]
            stbuf[1, pl.ds(f * 16, 16)] = carry[nf + f]
        pltpu.sync_copy(stbuf, st_hbm.at[wid])

    return k(p, q, dstc, srcc)


@functools.partial(jax.jit, static_argnames=("d",))
def _sc_scatter(h, dstc, *, d):
    hw = d // 2
    cpt = NCH // NS
    rpt = NPAD // NS

    @functools.partial(
        pl.kernel,
        out_type=jax.ShapeDtypeStruct((NPAD, d), jnp.float32),
        mesh=plsc.VectorSubcoreMesh(**_SC_MESH),
        scratch_types=[
            pltpu.VMEM((C,), jnp.int32),
            pltpu.VMEM((C, hw), jnp.float32),
            pltpu.VMEM((128, hw), jnp.float32),
            pltpu.VMEM_SHARED((NPAD, hw), jnp.float32),
        ],
    )
    def k(h_hbm, dst_hbm, out_hbm, idxbuf, rows, zbuf, acc):
        c = lax.axis_index("c")
        s = lax.axis_index("s")

        def zrow(r, _):
            for f in range(hw // 16):
                zbuf[r, pl.ds(f * 16, 16)] = jnp.zeros((16,), jnp.float32)
            return 0

        lax.fori_loop(0, 128, zrow, 0)
        r0 = s * rpt

        def zchunk(kk, _):
            pltpu.sync_copy(zbuf, acc.at[pl.ds(r0 + kk * 128, 128), :])
            return 0

        lax.fori_loop(0, rpt // 128, zchunk, 0)
        plsc.subcore_barrier()

        def chunk_body(i, _):
            j = s * cpt + i
            pltpu.sync_copy(dst_hbm.at[j], idxbuf)

            @pl.when(c == 0)
            def _():
                pltpu.sync_copy(h_hbm.at[pl.ds(j * C, C), pl.ds(0, hw)], rows)

            @pl.when(c == 1)
            def _():
                pltpu.sync_copy(h_hbm.at[pl.ds(j * C, C), pl.ds(hw, hw)], rows)

            pltpu.sync_copy(rows, acc.at[idxbuf], add=True)
            return 0

        lax.fori_loop(0, cpt, chunk_body, 0)
        plsc.subcore_barrier()

        def wchunk(kk, _):
            rr = r0 + kk * 128

            @pl.when(c == 0)
            def _():
                pltpu.sync_copy(acc.at[pl.ds(rr, 128), :],
                                out_hbm.at[pl.ds(rr, 128), pl.ds(0, hw)])

            @pl.when(c == 1)
            def _():
                pltpu.sync_copy(acc.at[pl.ds(rr, 128), :],
                                out_hbm.at[pl.ds(rr, 128), pl.ds(hw, hw)])

            return 0

        lax.fori_loop(0, rpt // 128, wchunk, 0)

    return k(h, dstc)


@jax.jit
def _sc_counts(dstc):
    cpt = NCH // NW
    rpt = NPAD // NS

    @functools.partial(
        pl.kernel,
        out_type=jax.ShapeDtypeStruct((NC, NPAD, 16), jnp.float32),
        mesh=plsc.VectorSubcoreMesh(**_SC_MESH),
        scratch_types=[
            pltpu.VMEM((C,), jnp.int32),
            pltpu.VMEM((C, 16), jnp.float32),
            pltpu.VMEM((128, 16), jnp.float32),
            pltpu.VMEM_SHARED((NPAD, 16), jnp.float32),
        ],
    )
    def k(dst_hbm, out_hbm, idxbuf, ones, zbuf, acc):
        c = lax.axis_index("c")
        s = lax.axis_index("s")

        def zrow(r, _):
            zbuf[r, pl.ds(0, 16)] = jnp.zeros((16,), jnp.float32)
            return 0

        def orow(r, _):
            ones[r, pl.ds(0, 16)] = jnp.ones((16,), jnp.float32)
            return 0

        lax.fori_loop(0, 128, zrow, 0)
        lax.fori_loop(0, C, orow, 0)
        r0 = s * rpt

        def zchunk(kk, _):
            pltpu.sync_copy(zbuf, acc.at[pl.ds(r0 + kk * 128, 128), :])
            return 0

        lax.fori_loop(0, rpt // 128, zchunk, 0)
        plsc.subcore_barrier()

        def chunk_body(i, _):
            j = (c * NS + s) * cpt + i
            pltpu.sync_copy(dst_hbm.at[j], idxbuf)
            pltpu.sync_copy(ones, acc.at[idxbuf], add=True)
            return 0

        lax.fori_loop(0, cpt, chunk_body, 0)
        plsc.subcore_barrier()

        def wchunk(kk, _):
            rr = r0 + kk * 128
            pltpu.sync_copy(acc.at[pl.ds(rr, 128), :],
                            out_hbm.at[c, pl.ds(rr, 128), :])
            return 0

        lax.fori_loop(0, rpt // 128, wchunk, 0)

    return k(dstc)


def _project_body(agg_ref, cnt_ref, a_ref, b_ref, p_ref, q_ref):
    cc = cnt_ref[...]
    cnt = cc[0, :, 0] + cc[1, :, 0]
    inv = 1.0 / jnp.maximum(cnt, 1.0)
    h = jax.nn.relu(agg_ref[...] * inv[:, None])
    p_ref[...] = jnp.dot(h, a_ref[...], preferred_element_type=jnp.float32)
    q_ref[...] = jnp.dot(h, b_ref[...], preferred_element_type=jnp.float32)


@jax.jit
def _tc_project(agg, cnt16, A, B):
    dp, dn = A.shape
    return pl.pallas_call(
        _project_body,
        grid=(NNB,),
        in_specs=[
            pl.BlockSpec((NB, dp), lambda i: (i, 0)),
            pl.BlockSpec((NC, NB, 16), lambda i: (0, i, 0)),
            pl.BlockSpec((dp, dn), lambda i: (0, 0)),
            pl.BlockSpec((dp, dn), lambda i: (0, 0)),
        ],
        out_specs=[
            pl.BlockSpec((NB, dn), lambda i: (i, 0)),
            pl.BlockSpec((NB, dn), lambda i: (i, 0)),
        ],
        out_shape=[
            jax.ShapeDtypeStruct((N, dn), jnp.float32),
            jax.ShapeDtypeStruct((N, dn), jnp.float32),
        ],
    )(agg, cnt16, A, B)


def _project1_body(x_ref, a_ref, b_ref, p_ref, q_ref):
    h = x_ref[...]
    p_ref[...] = jnp.dot(h, a_ref[...], preferred_element_type=jnp.float32)
    q_ref[...] = jnp.dot(h, b_ref[...], preferred_element_type=jnp.float32)


@jax.jit
def _tc_project1(x, A, B):
    dp, dn = A.shape
    return pl.pallas_call(
        _project1_body,
        grid=(NNB,),
        in_specs=[
            pl.BlockSpec((NB, dp), lambda i: (i, 0)),
            pl.BlockSpec((dp, dn), lambda i: (0, 0)),
            pl.BlockSpec((dp, dn), lambda i: (0, 0)),
        ],
        out_specs=[
            pl.BlockSpec((NB, dn), lambda i: (i, 0)),
            pl.BlockSpec((NB, dn), lambda i: (i, 0)),
        ],
        out_shape=[
            jax.ShapeDtypeStruct((N, dn), jnp.float32),
            jax.ShapeDtypeStruct((N, dn), jnp.float32),
        ],
    )(x, A, B)


def _bn_scale_shift(sp_ref, g_ref, b_ref):
    sp = jnp.sum(sp_ref[...], axis=0)
    mu = sp[0] * (1.0 / E)
    var = sp[1] * (1.0 / E) - mu * mu
    rstd = lax.rsqrt(var + EPS)
    scale = g_ref[...][0] * rstd
    shift = b_ref[...][0] - mu * scale
    return scale, shift


def _mid_body(sp_ref, y_ref, g_ref, b_ref, w_ref, yn_ref, st_ref):
    i = pl.program_id(0)
    scale, shift = _bn_scale_shift(sp_ref, g_ref, b_ref)
    h = jax.nn.relu(y_ref[...] * scale[None, :] + shift[None, :])
    yn = jnp.dot(h, w_ref[...], preferred_element_type=jnp.float32)
    yn_ref[...] = yn

    @pl.when(i == 0)
    def _():
        st_ref[...] = jnp.zeros_like(st_ref)

    st_ref[...] = st_ref[...] + jnp.stack(
        [jnp.sum(yn, axis=0), jnp.sum(yn * yn, axis=0)])


@jax.jit
def _tc_mid(Y, spart, gamma, beta, W):
    d = Y.shape[1]
    K = spart.shape[0]
    return pl.pallas_call(
        _mid_body,
        grid=(NEB,),
        in_specs=[
            pl.BlockSpec((K, 2, d), lambda i: (0, 0, 0)),
            pl.BlockSpec((EB, d), lambda i: (i, 0)),
            pl.BlockSpec((1, d), lambda i: (0, 0)),
            pl.BlockSpec((1, d), lambda i: (0, 0)),
            pl.BlockSpec((d, d), lambda i: (0, 0)),
        ],
        out_specs=[
            pl.BlockSpec((EB, d), lambda i: (i, 0)),
            pl.BlockSpec((2, d), lambda i: (0, 0)),
        ],
        out_shape=[
            jax.ShapeDtypeStruct((E, d), jnp.float32),
            jax.ShapeDtypeStruct((2, d), jnp.float32),
        ],
    )(spart, Y, gamma, beta, W)


def _last_body(sp_ref, y_ref, g_ref, b_ref, h_ref):
    scale, shift = _bn_scale_shift(sp_ref, g_ref, b_ref)
    h_ref[...] = jax.nn.relu(y_ref[...] * scale[None, :] + shift[None, :])


@jax.jit
def _tc_last(Y, spart, gamma, beta):
    d = Y.shape[1]
    K = spart.shape[0]
    return pl.pallas_call(
        _last_body,
        grid=(NEB,),
        in_specs=[
            pl.BlockSpec((K, 2, d), lambda i: (0, 0, 0)),
            pl.BlockSpec((EB, d), lambda i: (i, 0)),
            pl.BlockSpec((1, d), lambda i: (0, 0)),
            pl.BlockSpec((1, d), lambda i: (0, 0)),
        ],
        out_specs=pl.BlockSpec((EB, d), lambda i: (i, 0)),
        out_shape=jax.ShapeDtypeStruct((E, d), jnp.float32),
    )(spart, Y, gamma, beta)


def _final_body(agg_ref, cnt_ref, bvec_ref, bsm_ref, w1_ref, b1_ref,
                w2_ref, b2_ref, out_ref, pooled_ref):
    i = pl.program_id(0)

    @pl.when(i == 0)
    def _():
        pooled_ref[...] = jnp.full_like(pooled_ref, -jnp.inf)
        out_ref[...] = jnp.zeros_like(out_ref)

    cc = cnt_ref[...]
    cnt = cc[0, :, 0] + cc[1, :, 0]
    inv = 1.0 / jnp.maximum(cnt, 1.0)
    h = jax.nn.relu(agg_ref[...] * inv[:, None])
    bvec = bvec_ref[...]
    blo = bsm_ref[0]
    bhi = bsm_ref[NB - 1]
    for gc in range((G + 7) // 8):

        @pl.when((blo < 8 * gc + 8) & (bhi >= 8 * gc))
        def _():
            for g8 in range(8):
                g = gc * 8 + g8
                if g < G:
                    m = bvec == g
                    mv = jnp.max(jnp.where(m, h, -jnp.inf), axis=0)
                    pooled_ref[g, :] = jnp.maximum(pooled_ref[g, :], mv)

    @pl.when(i == NNB - 1)
    def _():
        p = pooled_ref[...]
        p = jnp.where(jnp.isfinite(p), p, 0.0)
        hh = jax.nn.relu(
            jnp.dot(p, w1_ref[...], preferred_element_type=jnp.float32)
            + b1_ref[...][0][None, :])
        out_ref[...] = (
            jnp.dot(hh, w2_ref[...], preferred_element_type=jnp.float32)
            + b2_ref[...][0][None, :])


@jax.jit
def _tc_final(agg, cnt16, bvec, batch, w1, b1, w2, b2):
    d = agg.shape[1]
    no = w2.shape[1]
    return pl.pallas_call(
        _final_body,
        grid=(NNB,),
        in_specs=[
            pl.BlockSpec((NB, d), lambda i: (i, 0)),
            pl.BlockSpec((NC, NB, 16), lambda i: (0, i, 0)),
            pl.BlockSpec((NB, 1), lambda i: (i, 0)),
            pl.BlockSpec((NB,), lambda i: (i,), memory_space=pltpu.SMEM),
            pl.BlockSpec((d, d), lambda i: (0, 0)),
            pl.BlockSpec((1, d), lambda i: (0, 0)),
            pl.BlockSpec((d, no), lambda i: (0, 0)),
            pl.BlockSpec((1, no), lambda i: (0, 0)),
        ],
        out_specs=pl.BlockSpec((G, no), lambda i: (0, 0)),
        out_shape=jax.ShapeDtypeStruct((G, no), jnp.float32),
        scratch_shapes=[pltpu.VMEM((G, d), jnp.float32)],
    )(agg, cnt16, bvec, batch, w1, b1, w2, b2)


def _conv(h_nodes, cnt16, dstc, srcc, layers, first):
    dp = layers[0]["W"].shape[0] // 2
    dn = layers[0]["W"].shape[1]
    W1 = layers[0]["W"]
    A = W1[:dp] - W1[dp:]
    B = W1[dp:]
    if first:
        P, Q = _tc_project1(h_nodes, A, B)
    else:
        P, Q = _tc_project(h_nodes, cnt16, A, B)
    Y1, sp1 = _sc_gather_add(P, Q, dstc, srcc, d=dn)
    g = [lyr["gamma"].reshape(1, dn) for lyr in layers]
    b = [lyr["beta"].reshape(1, dn) for lyr in layers]
    Y2, s2 = _tc_mid(Y1, sp1, g[0], b[0], layers[1]["W"])
    Y3, s3 = _tc_mid(Y2, s2[None], g[1], b[1], layers[2]["W"])
    H = _tc_last(Y3, s3[None], g[2], b[2])
    return _sc_scatter(H, dstc, d=dn)


def kernel(x, edge_index, batch, params):
    srcc = edge_index[0].reshape(NCH, C)
    dstc = edge_index[1].reshape(NCH, C)
    cnt16 = _sc_counts(dstc)
    agg = _conv(x, cnt16, dstc, srcc, params["conv1"], first=True)
    agg = _conv(agg, cnt16, dstc, srcc, params["conv2"], first=False)
    agg = _conv(agg, cnt16, dstc, srcc, params["conv3"], first=False)
    return _tc_final(
        agg, cnt16, batch.reshape(N, 1), batch,
        params["fc1_W"], params["fc1_b"].reshape(1, -1),
        params["out_W"], params["out_b"].reshape(1, -1))


# speedup vs baseline: 2.1769x; 2.1769x over previous
import functools

import jax
import jax.numpy as jnp
from jax import lax
from jax.experimental import pallas as pl
from jax.experimental.pallas import tpu as pltpu
import jax.experimental.pallas.tpu_sc as plsc

N = 10000
NPAD = 10240
E = 160000
G = 100
EPS = 1e-5
C = 128
NCH = E // C
NC, NS = 2, 16
NW = NC * NS
EB = 2000
NEB = E // EB
NB = 1000
NNB = N // NB

_SC_MESH = dict(core_axis_name="c", subcore_axis_name="s")


@functools.partial(jax.jit, static_argnames=("d",))
def _sc_gather_add(p, q, dstc, srcc, *, d):
    nf = d // 16

    @functools.partial(
        pl.kernel,
        out_type=(
            jax.ShapeDtypeStruct((E, d), jnp.float32),
            jax.ShapeDtypeStruct((NW, 2, d), jnp.float32),
        ),
        mesh=plsc.VectorSubcoreMesh(**_SC_MESH),
        scratch_types=[
            pltpu.VMEM((C,), jnp.int32),
            pltpu.VMEM((C,), jnp.int32),
            pltpu.VMEM((C, d), jnp.float32),
            pltpu.VMEM((C, d), jnp.float32),
            pltpu.VMEM((2, d), jnp.float32),
            pltpu.SemaphoreType.DMA,
            pltpu.SemaphoreType.DMA,
        ],
    )
    def k(p_hbm, q_hbm, dst_hbm, src_hbm, y_hbm, st_hbm,
          dbuf, sbuf, pbuf, qbuf, stbuf, sem1, sem2):
        wid = lax.axis_index("s") * NC + lax.axis_index("c")
        lo = wid * NCH // NW
        hi = (wid + 1) * NCH // NW
        zero = jnp.zeros((16,), jnp.float32)
        init = tuple(zero for _ in range(2 * nf))

        def chunk_body(j, carry):
            pltpu.sync_copy(dst_hbm.at[j], dbuf)
            pltpu.sync_copy(src_hbm.at[j], sbuf)
            cp1 = pltpu.async_copy(p_hbm.at[dbuf], pbuf, sem1)
            cp2 = pltpu.async_copy(q_hbm.at[sbuf], qbuf, sem2)
            cp1.wait()
            cp2.wait()

            def row_body(r, rc):
                acc = list(rc)
                for f in range(nf):
                    y = pbuf[r, pl.ds(f * 16, 16)] + qbuf[r, pl.ds(f * 16, 16)]
                    pbuf[r, pl.ds(f * 16, 16)] = y
                    acc[f] = acc[f] + y
                    acc[nf + f] = acc[nf + f] + y * y
                return tuple(acc)

            carry = lax.fori_loop(0, C, row_body, carry)
            pltpu.sync_copy(pbuf, y_hbm.at[pl.ds(j * C, C), :])
            return carry

        carry = lax.fori_loop(lo, hi, chunk_body, init)
        for f in range(nf):
            stbuf[0, pl.ds(f * 16, 16)] = carry[f]
            stbuf[1, pl.ds(f * 16, 16)] = carry[nf + f]
        pltpu.sync_copy(stbuf, st_hbm.at[wid])

    return k(p, q, dstc, srcc)


@functools.partial(jax.jit, static_argnames=("d",))
def _sc_scatter(h, dstc, *, d):
    split_feat = d > 128
    hw = d // 2 if split_feat else d
    rpt = NPAD // NS
    out_ty = (jax.ShapeDtypeStruct((NPAD, d), jnp.float32) if split_feat
              else jax.ShapeDtypeStruct((NC, NPAD, d), jnp.float32))

    @functools.partial(
        pl.kernel,
        out_type=out_ty,
        mesh=plsc.VectorSubcoreMesh(**_SC_MESH),
        scratch_types=[
            pltpu.VMEM((C,), jnp.int32),
            pltpu.VMEM((C, hw), jnp.float32),
            pltpu.VMEM((128, hw), jnp.float32),
            pltpu.VMEM_SHARED((NPAD, hw), jnp.float32),
        ],
    )
    def k(h_hbm, dst_hbm, out_hbm, idxbuf, rows, zbuf, acc):
        c = lax.axis_index("c")
        s = lax.axis_index("s")

        def zrow(r, _):
            for f in range(hw // 16):
                zbuf[r, pl.ds(f * 16, 16)] = jnp.zeros((16,), jnp.float32)
            return 0

        lax.fori_loop(0, 128, zrow, 0)
        r0 = s * rpt

        def zchunk(kk, _):
            pltpu.sync_copy(zbuf, acc.at[pl.ds(r0 + kk * 128, 128), :])
            return 0

        lax.fori_loop(0, rpt // 128, zchunk, 0)
        plsc.subcore_barrier()

        if split_feat:
            lo = s * NCH // NS
            hi = (s + 1) * NCH // NS
        else:
            lo = (c * NS + s) * NCH // NW
            hi = (c * NS + s + 1) * NCH // NW

        def chunk_body(j, _):
            pltpu.sync_copy(dst_hbm.at[j], idxbuf)
            if split_feat:
                @pl.when(c == 0)
                def _():
                    pltpu.sync_copy(h_hbm.at[pl.ds(j * C, C), pl.ds(0, hw)],
                                    rows)

                @pl.when(c == 1)
                def _():
                    pltpu.sync_copy(h_hbm.at[pl.ds(j * C, C), pl.ds(hw, hw)],
                                    rows)
            else:
                pltpu.sync_copy(h_hbm.at[pl.ds(j * C, C), :], rows)
            pltpu.sync_copy(rows, acc.at[idxbuf], add=True)
            return 0

        lax.fori_loop(lo, hi, chunk_body, 0)
        plsc.subcore_barrier()

        def wchunk(kk, _):
            rr = r0 + kk * 128
            if split_feat:
                @pl.when(c == 0)
                def _():
                    pltpu.sync_copy(acc.at[pl.ds(rr, 128), :],
                                    out_hbm.at[pl.ds(rr, 128), pl.ds(0, hw)])

                @pl.when(c == 1)
                def _():
                    pltpu.sync_copy(acc.at[pl.ds(rr, 128), :],
                                    out_hbm.at[pl.ds(rr, 128), pl.ds(hw, hw)])
            else:
                pltpu.sync_copy(acc.at[pl.ds(rr, 128), :],
                                out_hbm.at[c, pl.ds(rr, 128), :])
            return 0

        lax.fori_loop(0, rpt // 128, wchunk, 0)

    return k(h, dstc)


@jax.jit
def _sc_counts(dstc):
    rpt = NPAD // NS

    @functools.partial(
        pl.kernel,
        out_type=jax.ShapeDtypeStruct((NC, NPAD, 128), jnp.float32),
        mesh=plsc.VectorSubcoreMesh(**_SC_MESH),
        scratch_types=[
            pltpu.VMEM((C,), jnp.int32),
            pltpu.VMEM((C, 128), jnp.float32),
            pltpu.VMEM((128, 128), jnp.float32),
            pltpu.VMEM_SHARED((NPAD, 128), jnp.float32),
        ],
    )
    def k(dst_hbm, out_hbm, idxbuf, ones, zbuf, acc):
        c = lax.axis_index("c")
        s = lax.axis_index("s")

        def zrow(r, _):
            for f in range(8):
                zbuf[r, pl.ds(f * 16, 16)] = jnp.zeros((16,), jnp.float32)
            return 0

        def orow(r, _):
            for f in range(8):
                ones[r, pl.ds(f * 16, 16)] = jnp.ones((16,), jnp.float32)
            return 0

        lax.fori_loop(0, 128, zrow, 0)
        lax.fori_loop(0, C, orow, 0)
        r0 = s * rpt

        def zchunk(kk, _):
            pltpu.sync_copy(zbuf, acc.at[pl.ds(r0 + kk * 128, 128), :])
            return 0

        lax.fori_loop(0, rpt // 128, zchunk, 0)
        plsc.subcore_barrier()

        lo = (c * NS + s) * NCH // NW
        hi = (c * NS + s + 1) * NCH // NW

        def chunk_body(j, _):
            pltpu.sync_copy(dst_hbm.at[j], idxbuf)
            pltpu.sync_copy(ones, acc.at[idxbuf], add=True)
            return 0

        lax.fori_loop(lo, hi, chunk_body, 0)
        plsc.subcore_barrier()

        def wchunk(kk, _):
            rr = r0 + kk * 128
            pltpu.sync_copy(acc.at[pl.ds(rr, 128), :],
                            out_hbm.at[c, pl.ds(rr, 128), :])
            return 0

        lax.fori_loop(0, rpt // 128, wchunk, 0)

    return k(dstc)


def _project_body(agg_ref, cnt_ref, a_ref, b_ref, p_ref, q_ref):
    cc = cnt_ref[...]
    cnt = cc[0, :, 0] + cc[1, :, 0]
    inv = 1.0 / jnp.maximum(cnt, 1.0)
    aa = agg_ref[...]
    h = jax.nn.relu((aa[0] + aa[1]) * inv[:, None])
    p_ref[...] = jnp.dot(h, a_ref[...], preferred_element_type=jnp.float32, precision=lax.Precision.HIGHEST)
    q_ref[...] = jnp.dot(h, b_ref[...], preferred_element_type=jnp.float32, precision=lax.Precision.HIGHEST)


@jax.jit
def _tc_project(agg, cnt16, A, B):
    dp, dn = A.shape
    return pl.pallas_call(
        _project_body,
        grid=(NNB,),
        in_specs=[
            pl.BlockSpec((NC, NB, dp), lambda i: (0, i, 0)),
            pl.BlockSpec((NC, NB, 128), lambda i: (0, i, 0)),
            pl.BlockSpec((dp, dn), lambda i: (0, 0)),
            pl.BlockSpec((dp, dn), lambda i: (0, 0)),
        ],
        out_specs=[
            pl.BlockSpec((NB, dn), lambda i: (i, 0)),
            pl.BlockSpec((NB, dn), lambda i: (i, 0)),
        ],
        out_shape=[
            jax.ShapeDtypeStruct((N, dn), jnp.float32),
            jax.ShapeDtypeStruct((N, dn), jnp.float32),
        ],
    )(agg, cnt16, A, B)


def _project1_body(x_ref, a_ref, b_ref, p_ref, q_ref):
    h = x_ref[...]
    p_ref[...] = jnp.dot(h, a_ref[...], preferred_element_type=jnp.float32, precision=lax.Precision.HIGHEST)
    q_ref[...] = jnp.dot(h, b_ref[...], preferred_element_type=jnp.float32, precision=lax.Precision.HIGHEST)


@jax.jit
def _tc_project1(x, A, B):
    dp, dn = A.shape
    return pl.pallas_call(
        _project1_body,
        grid=(NNB,),
        in_specs=[
            pl.BlockSpec((NB, dp), lambda i: (i, 0)),
            pl.BlockSpec((dp, dn), lambda i: (0, 0)),
            pl.BlockSpec((dp, dn), lambda i: (0, 0)),
        ],
        out_specs=[
            pl.BlockSpec((NB, dn), lambda i: (i, 0)),
            pl.BlockSpec((NB, dn), lambda i: (i, 0)),
        ],
        out_shape=[
            jax.ShapeDtypeStruct((N, dn), jnp.float32),
            jax.ShapeDtypeStruct((N, dn), jnp.float32),
        ],
    )(x, A, B)


def _bn_scale_shift(sp_ref, g_ref, b_ref):
    sp = jnp.sum(sp_ref[...], axis=0)
    mu = sp[0] * (1.0 / E)
    var = sp[1] * (1.0 / E) - mu * mu
    rstd = lax.rsqrt(var + EPS)
    scale = g_ref[...][0] * rstd
    shift = b_ref[...][0] - mu * scale
    return scale, shift


def _mid_body(sp_ref, y_ref, g_ref, b_ref, w_ref, yn_ref, st_ref):
    i = pl.program_id(0)
    scale, shift = _bn_scale_shift(sp_ref, g_ref, b_ref)
    h = jax.nn.relu(y_ref[...] * scale[None, :] + shift[None, :])
    yn = jnp.dot(h, w_ref[...], preferred_element_type=jnp.float32, precision=lax.Precision.HIGHEST)
    yn_ref[...] = yn

    @pl.when(i == 0)
    def _():
        st_ref[...] = jnp.zeros_like(st_ref)

    st_ref[...] = st_ref[...] + jnp.stack(
        [jnp.sum(yn, axis=0), jnp.sum(yn * yn, axis=0)])


@jax.jit
def _tc_mid(Y, spart, gamma, beta, W):
    d = Y.shape[1]
    K = spart.shape[0]
    return pl.pallas_call(
        _mid_body,
        grid=(NEB,),
        in_specs=[
            pl.BlockSpec((K, 2, d), lambda i: (0, 0, 0)),
            pl.BlockSpec((EB, d), lambda i: (i, 0)),
            pl.BlockSpec((1, d), lambda i: (0, 0)),
            pl.BlockSpec((1, d), lambda i: (0, 0)),
            pl.BlockSpec((d, d), lambda i: (0, 0)),
        ],
        out_specs=[
            pl.BlockSpec((EB, d), lambda i: (i, 0)),
            pl.BlockSpec((2, d), lambda i: (0, 0)),
        ],
        out_shape=[
            jax.ShapeDtypeStruct((E, d), jnp.float32),
            jax.ShapeDtypeStruct((2, d), jnp.float32),
        ],
    )(spart, Y, gamma, beta, W)


def _last_body(sp_ref, y_ref, g_ref, b_ref, h_ref):
    scale, shift = _bn_scale_shift(sp_ref, g_ref, b_ref)
    h_ref[...] = jax.nn.relu(y_ref[...] * scale[None, :] + shift[None, :])


@jax.jit
def _tc_last(Y, spart, gamma, beta):
    d = Y.shape[1]
    K = spart.shape[0]
    return pl.pallas_call(
        _last_body,
        grid=(NEB,),
        in_specs=[
            pl.BlockSpec((K, 2, d), lambda i: (0, 0, 0)),
            pl.BlockSpec((EB, d), lambda i: (i, 0)),
            pl.BlockSpec((1, d), lambda i: (0, 0)),
            pl.BlockSpec((1, d), lambda i: (0, 0)),
        ],
        out_specs=pl.BlockSpec((EB, d), lambda i: (i, 0)),
        out_shape=jax.ShapeDtypeStruct((E, d), jnp.float32),
    )(spart, Y, gamma, beta)


def _final_body(agg_ref, cnt_ref, bvec_ref, bsm_ref, w1_ref, b1_ref,
                w2_ref, b2_ref, out_ref, pooled_ref):
    i = pl.program_id(0)

    @pl.when(i == 0)
    def _():
        pooled_ref[...] = jnp.full_like(pooled_ref, -jnp.inf)
        out_ref[...] = jnp.zeros_like(out_ref)

    cc = cnt_ref[...]
    cnt = cc[0, :, 0] + cc[1, :, 0]
    inv = 1.0 / jnp.maximum(cnt, 1.0)
    h = jax.nn.relu(agg_ref[...] * inv[:, None])
    bvec = bvec_ref[...]
    blo = bsm_ref[i * NB]
    bhi = bsm_ref[i * NB + NB - 1]
    for gc in range((G + 7) // 8):

        @pl.when((blo < 8 * gc + 8) & (bhi >= 8 * gc))
        def _():
            for g8 in range(8):
                g = gc * 8 + g8
                if g < G:
                    m = bvec == g
                    mv = jnp.max(jnp.where(m, h, -jnp.inf), axis=0)
                    pooled_ref[g, :] = jnp.maximum(pooled_ref[g, :], mv)

    @pl.when(i == NNB - 1)
    def _():
        p = pooled_ref[...]
        p = jnp.where(jnp.isfinite(p), p, 0.0)
        hh = jax.nn.relu(
            jnp.dot(p, w1_ref[...], preferred_element_type=jnp.float32, precision=lax.Precision.HIGHEST)
            + b1_ref[...][0][None, :])
        out_ref[...] = (
            jnp.dot(hh, w2_ref[...], preferred_element_type=jnp.float32, precision=lax.Precision.HIGHEST)
            + b2_ref[...][0][None, :])


@jax.jit
def _tc_final(agg, cnt16, bvec, batch, w1, b1, w2, b2):
    d = agg.shape[1]
    no = w2.shape[1]
    return pl.pallas_call(
        _final_body,
        grid=(NNB,),
        in_specs=[
            pl.BlockSpec((NB, d), lambda i: (i, 0)),
            pl.BlockSpec((NC, NB, 128), lambda i: (0, i, 0)),
            pl.BlockSpec((NB, 1), lambda i: (i, 0)),
            pl.BlockSpec((N,), lambda i: (0,), memory_space=pltpu.SMEM),
            pl.BlockSpec((d, d), lambda i: (0, 0)),
            pl.BlockSpec((1, d), lambda i: (0, 0)),
            pl.BlockSpec((d, no), lambda i: (0, 0)),
            pl.BlockSpec((1, no), lambda i: (0, 0)),
        ],
        out_specs=pl.BlockSpec((G, no), lambda i: (0, 0)),
        out_shape=jax.ShapeDtypeStruct((G, no), jnp.float32),
        scratch_shapes=[pltpu.VMEM((G, d), jnp.float32)],
    )(agg, cnt16, bvec, batch, w1, b1, w2, b2)


def _pad_to(a, rows, cols):
    return jnp.pad(a, ((0, rows - a.shape[0]), (0, cols - a.shape[1])))


def _conv(h_nodes, cnt16, dstc, srcc, layers, dg, first):
    dp = layers[0]["W"].shape[0] // 2
    dn = layers[0]["W"].shape[1]
    W1 = layers[0]["W"]
    A = _pad_to(W1[:dp] - W1[dp:], 128 if not first else dp, dg)
    B = _pad_to(W1[dp:], 128 if not first else dp, dg)
    if first:
        P, Q = _tc_project1(h_nodes, A, B)
    else:
        P, Q = _tc_project(h_nodes, cnt16, A, B)
    Y1, sp1 = _sc_gather_add(P, Q, dstc, srcc, d=dg)
    g = [_pad_to(lyr["gamma"].reshape(1, dn), 1, dg) for lyr in layers]
    b = [_pad_to(lyr["beta"].reshape(1, dn), 1, dg) for lyr in layers]
    W2 = _pad_to(layers[1]["W"], dg, dg)
    W3 = _pad_to(layers[2]["W"], dg, dg)
    Y2, s2 = _tc_mid(Y1, sp1, g[0], b[0], W2)
    Y3, s3 = _tc_mid(Y2, s2[None], g[1], b[1], W3)
    H = _tc_last(Y3, s3[None], g[2], b[2])
    return _sc_scatter(H, dstc, d=dg)


def kernel(x, edge_index, batch, params):
    srcc = edge_index[0].reshape(NCH, C)
    dstc = edge_index[1].reshape(NCH, C)
    cnt16 = _sc_counts(dstc)
    agg = _conv(x, cnt16, dstc, srcc, params["conv1"], 128, first=True)
    agg = _conv(agg, cnt16, dstc, srcc, params["conv2"], 128, first=False)
    agg = _conv(agg, cnt16, dstc, srcc, params["conv3"], 256, first=False)
    return _tc_final(
        agg, cnt16, batch.reshape(N, 1), batch,
        params["fc1_W"], params["fc1_b"].reshape(1, -1),
        params["out_W"], params["out_b"].reshape(1, -1))


# speedup vs baseline: 2.3294x; 1.0700x over previous
import functools

import jax
import jax.numpy as jnp
from jax import lax
from jax.experimental import pallas as pl
from jax.experimental.pallas import tpu as pltpu
import jax.experimental.pallas.tpu_sc as plsc

N = 10000
NPAD = 10240
E = 160000
G = 100
EPS = 1e-5
C = 128
NCH = E // C
NC, NS = 2, 16
NW = NC * NS
EB = 2000
NEB = E // EB
NB = 1000
NNB = N // NB

_SC_MESH = dict(core_axis_name="c", subcore_axis_name="s")


@functools.partial(jax.jit, static_argnames=("d",))
def _sc_gather_add(p, q, dstc, srcc, *, d):
    nf = d // 16

    @functools.partial(
        pl.kernel,
        out_type=(
            jax.ShapeDtypeStruct((E, d), jnp.float32),
            jax.ShapeDtypeStruct((NW, 2, d), jnp.float32),
        ),
        mesh=plsc.VectorSubcoreMesh(**_SC_MESH),
        scratch_types=[
            pltpu.VMEM((C,), jnp.int32),
            pltpu.VMEM((C,), jnp.int32),
            pltpu.VMEM((C, d), jnp.float32),
            pltpu.VMEM((C, d), jnp.float32),
            pltpu.VMEM((2, d), jnp.float32),
            pltpu.SemaphoreType.DMA,
            pltpu.SemaphoreType.DMA,
        ],
    )
    def k(p_hbm, q_hbm, dst_hbm, src_hbm, y_hbm, st_hbm,
          dbuf, sbuf, pbuf, qbuf, stbuf, sem1, sem2):
        wid = lax.axis_index("s") * NC + lax.axis_index("c")
        lo = wid * NCH // NW
        hi = (wid + 1) * NCH // NW
        zero = jnp.zeros((16,), jnp.float32)
        init = tuple(zero for _ in range(2 * nf))

        def chunk_body(j, carry):
            pltpu.sync_copy(dst_hbm.at[j], dbuf)
            pltpu.sync_copy(src_hbm.at[j], sbuf)
            cp1 = pltpu.async_copy(p_hbm.at[dbuf], pbuf, sem1)
            cp2 = pltpu.async_copy(q_hbm.at[sbuf], qbuf, sem2)
            cp1.wait()
            cp2.wait()

            def row_body(r, rc):
                acc = list(rc)
                for f in range(nf):
                    y = pbuf[r, pl.ds(f * 16, 16)] + qbuf[r, pl.ds(f * 16, 16)]
                    pbuf[r, pl.ds(f * 16, 16)] = y
                    acc[f] = acc[f] + y
                    acc[nf + f] = acc[nf + f] + y * y
                return tuple(acc)

            carry = lax.fori_loop(0, C, row_body, carry)
            pltpu.sync_copy(pbuf, y_hbm.at[pl.ds(j * C, C), :])
            return carry

        carry = lax.fori_loop(lo, hi, chunk_body, init)
        for f in range(nf):
            stbuf[0, pl.ds(f * 16, 16)] = carry[f]
            stbuf[1, pl.ds(f * 16, 16)] = carry[nf + f]
        pltpu.sync_copy(stbuf, st_hbm.at[wid])

    return k(p, q, dstc, srcc)


@functools.partial(jax.jit, static_argnames=("d",))
def _sc_scatter(h, ss, dstc, *, d):
    split_feat = d > 128
    hw = d // 2 if split_feat else d
    nf = hw // 16
    rpt = NPAD // NS
    out_ty = (jax.ShapeDtypeStruct((NPAD, d), jnp.float32) if split_feat
              else jax.ShapeDtypeStruct((NC, NPAD, d), jnp.float32))

    @functools.partial(
        pl.kernel,
        out_type=out_ty,
        mesh=plsc.VectorSubcoreMesh(**_SC_MESH),
        scratch_types=[
            pltpu.VMEM((C,), jnp.int32),
            pltpu.VMEM((C, hw), jnp.float32),
            pltpu.VMEM((128, hw), jnp.float32),
            pltpu.VMEM((2, hw), jnp.float32),
            pltpu.VMEM_SHARED((NPAD, hw), jnp.float32),
        ],
    )
    def k(h_hbm, ss_hbm, dst_hbm, out_hbm, idxbuf, rows, zbuf, ssbuf, acc):
        c = lax.axis_index("c")
        s = lax.axis_index("s")
        pltpu.sync_copy(ss_hbm.at[c], ssbuf)
        sv = [ssbuf[0, pl.ds(f * 16, 16)] for f in range(nf)]
        tv = [ssbuf[1, pl.ds(f * 16, 16)] for f in range(nf)]

        def zrow(r, _):
            for f in range(hw // 16):
                zbuf[r, pl.ds(f * 16, 16)] = jnp.zeros((16,), jnp.float32)
            return 0

        lax.fori_loop(0, 128, zrow, 0)
        r0 = s * rpt

        def zchunk(kk, _):
            pltpu.sync_copy(zbuf, acc.at[pl.ds(r0 + kk * 128, 128), :])
            return 0

        lax.fori_loop(0, rpt // 128, zchunk, 0)
        plsc.subcore_barrier()

        if split_feat:
            lo = s * NCH // NS
            hi = (s + 1) * NCH // NS
        else:
            lo = (c * NS + s) * NCH // NW
            hi = (c * NS + s + 1) * NCH // NW

        def chunk_body(j, _):
            pltpu.sync_copy(dst_hbm.at[j], idxbuf)
            if split_feat:
                @pl.when(c == 0)
                def _():
                    pltpu.sync_copy(h_hbm.at[pl.ds(j * C, C), pl.ds(0, hw)],
                                    rows)

                @pl.when(c == 1)
                def _():
                    pltpu.sync_copy(h_hbm.at[pl.ds(j * C, C), pl.ds(hw, hw)],
                                    rows)
            else:
                pltpu.sync_copy(h_hbm.at[pl.ds(j * C, C), :], rows)

            def row_body(r, _):
                for f in range(nf):
                    v = rows[r, pl.ds(f * 16, 16)]
                    rows[r, pl.ds(f * 16, 16)] = jnp.maximum(
                        v * sv[f] + tv[f], 0.0)
                return 0

            lax.fori_loop(0, C, row_body, 0)
            pltpu.sync_copy(rows, acc.at[idxbuf], add=True)
            return 0

        lax.fori_loop(lo, hi, chunk_body, 0)
        plsc.subcore_barrier()

        def wchunk(kk, _):
            rr = r0 + kk * 128
            if split_feat:
                @pl.when(c == 0)
                def _():
                    pltpu.sync_copy(acc.at[pl.ds(rr, 128), :],
                                    out_hbm.at[pl.ds(rr, 128), pl.ds(0, hw)])

                @pl.when(c == 1)
                def _():
                    pltpu.sync_copy(acc.at[pl.ds(rr, 128), :],
                                    out_hbm.at[pl.ds(rr, 128), pl.ds(hw, hw)])
            else:
                pltpu.sync_copy(acc.at[pl.ds(rr, 128), :],
                                out_hbm.at[c, pl.ds(rr, 128), :])
            return 0

        lax.fori_loop(0, rpt // 128, wchunk, 0)

    return k(h, ss, dstc)


@functools.partial(jax.jit, static_argnames=("d", "ones_col"))
def _tc_scaleshift(spart, gamma, beta, *, d, ones_col):
    hw = d // 2 if d > 128 else d

    def body(sp_ref, g_ref, b_ref, out_ref):
        scale, shift = _bn_scale_shift(sp_ref, g_ref, b_ref)
        if ones_col:
            ci = lax.iota(jnp.int32, d)
            scale = jnp.where(ci == 64, 0.0, scale)
            shift = jnp.where(ci == 64, 1.0, shift)
        ss = jnp.stack([scale, shift])
        if d > 128:
            out_ref[...] = jnp.stack([ss[:, :hw], ss[:, hw:]])
        else:
            out_ref[...] = jnp.stack([ss, ss])

    K = spart.shape[0]
    return pl.pallas_call(
        body,
        grid=(1,),
        in_specs=[
            pl.BlockSpec((K, 2, d), lambda i: (0, 0, 0)),
            pl.BlockSpec((1, d), lambda i: (0, 0)),
            pl.BlockSpec((1, d), lambda i: (0, 0)),
        ],
        out_specs=pl.BlockSpec((NC, 2, hw), lambda i: (0, 0, 0)),
        out_shape=jax.ShapeDtypeStruct((NC, 2, hw), jnp.float32),
    )(spart, gamma, beta)


def _project_body(agg_ref, cnt_ref, a_ref, b_ref, p_ref, q_ref):
    cc = cnt_ref[...]
    cnt = cc[0, :, 64] + cc[1, :, 64]
    inv = 1.0 / jnp.maximum(cnt, 1.0)
    aa = agg_ref[...]
    h = jax.nn.relu((aa[0] + aa[1]) * inv[:, None])
    p_ref[...] = jnp.dot(h, a_ref[...], preferred_element_type=jnp.float32, precision=lax.Precision.HIGHEST)
    q_ref[...] = jnp.dot(h, b_ref[...], preferred_element_type=jnp.float32, precision=lax.Precision.HIGHEST)


@jax.jit
def _tc_project(agg, cnt16, A, B):
    dp, dn = A.shape
    return pl.pallas_call(
        _project_body,
        grid=(NNB,),
        in_specs=[
            pl.BlockSpec((NC, NB, dp), lambda i: (0, i, 0)),
            pl.BlockSpec((NC, NB, 128), lambda i: (0, i, 0)),
            pl.BlockSpec((dp, dn), lambda i: (0, 0)),
            pl.BlockSpec((dp, dn), lambda i: (0, 0)),
        ],
        out_specs=[
            pl.BlockSpec((NB, dn), lambda i: (i, 0)),
            pl.BlockSpec((NB, dn), lambda i: (i, 0)),
        ],
        out_shape=[
            jax.ShapeDtypeStruct((N, dn), jnp.float32),
            jax.ShapeDtypeStruct((N, dn), jnp.float32),
        ],
    )(agg, cnt16, A, B)


def _project1_body(x_ref, a_ref, b_ref, p_ref, q_ref):
    h = x_ref[...]
    p_ref[...] = jnp.dot(h, a_ref[...], preferred_element_type=jnp.float32, precision=lax.Precision.HIGHEST)
    q_ref[...] = jnp.dot(h, b_ref[...], preferred_element_type=jnp.float32, precision=lax.Precision.HIGHEST)


@jax.jit
def _tc_project1(x, A, B):
    dp, dn = A.shape
    return pl.pallas_call(
        _project1_body,
        grid=(NNB,),
        in_specs=[
            pl.BlockSpec((NB, dp), lambda i: (i, 0)),
            pl.BlockSpec((dp, dn), lambda i: (0, 0)),
            pl.BlockSpec((dp, dn), lambda i: (0, 0)),
        ],
        out_specs=[
            pl.BlockSpec((NB, dn), lambda i: (i, 0)),
            pl.BlockSpec((NB, dn), lambda i: (i, 0)),
        ],
        out_shape=[
            jax.ShapeDtypeStruct((N, dn), jnp.float32),
            jax.ShapeDtypeStruct((N, dn), jnp.float32),
        ],
    )(x, A, B)


def _bn_scale_shift(sp_ref, g_ref, b_ref):
    sp = jnp.sum(sp_ref[...], axis=0)
    mu = sp[0] * (1.0 / E)
    var = sp[1] * (1.0 / E) - mu * mu
    rstd = lax.rsqrt(var + EPS)
    scale = g_ref[...][0] * rstd
    shift = b_ref[...][0] - mu * scale
    return scale, shift


def _mid_body(sp_ref, y_ref, g_ref, b_ref, w_ref, yn_ref, st_ref):
    i = pl.program_id(0)
    scale, shift = _bn_scale_shift(sp_ref, g_ref, b_ref)
    h = jax.nn.relu(y_ref[...] * scale[None, :] + shift[None, :])
    yn = jnp.dot(h, w_ref[...], preferred_element_type=jnp.float32, precision=lax.Precision.HIGHEST)
    yn_ref[...] = yn

    @pl.when(i == 0)
    def _():
        st_ref[...] = jnp.zeros_like(st_ref)

    st_ref[...] = st_ref[...] + jnp.stack(
        [jnp.sum(yn, axis=0), jnp.sum(yn * yn, axis=0)])


@jax.jit
def _tc_mid(Y, spart, gamma, beta, W):
    d = Y.shape[1]
    K = spart.shape[0]
    return pl.pallas_call(
        _mid_body,
        grid=(NEB,),
        in_specs=[
            pl.BlockSpec((K, 2, d), lambda i: (0, 0, 0)),
            pl.BlockSpec((EB, d), lambda i: (i, 0)),
            pl.BlockSpec((1, d), lambda i: (0, 0)),
            pl.BlockSpec((1, d), lambda i: (0, 0)),
            pl.BlockSpec((d, d), lambda i: (0, 0)),
        ],
        out_specs=[
            pl.BlockSpec((EB, d), lambda i: (i, 0)),
            pl.BlockSpec((2, d), lambda i: (0, 0)),
        ],
        out_shape=[
            jax.ShapeDtypeStruct((E, d), jnp.float32),
            jax.ShapeDtypeStruct((2, d), jnp.float32),
        ],
    )(spart, Y, gamma, beta, W)


def _final_body(agg_ref, cnt_ref, bvec_ref, bsm_ref, w1_ref, b1_ref,
                w2_ref, b2_ref, out_ref, pooled_ref):
    i = pl.program_id(0)

    @pl.when(i == 0)
    def _():
        pooled_ref[...] = jnp.full_like(pooled_ref, -jnp.inf)
        out_ref[...] = jnp.zeros_like(out_ref)

    cc = cnt_ref[...]
    cnt = cc[0, :, 64] + cc[1, :, 64]
    inv = 1.0 / jnp.maximum(cnt, 1.0)
    h = jax.nn.relu(agg_ref[...] * inv[:, None])
    bvec = bvec_ref[...]
    blo = bsm_ref[i * NB]
    bhi = bsm_ref[i * NB + NB - 1]
    for gc in range((G + 7) // 8):

        @pl.when((blo < 8 * gc + 8) & (bhi >= 8 * gc))
        def _():
            for g8 in range(8):
                g = gc * 8 + g8
                if g < G:
                    m = bvec == g
                    mv = jnp.max(jnp.where(m, h, -jnp.inf), axis=0)
                    pooled_ref[g, :] = jnp.maximum(pooled_ref[g, :], mv)

    @pl.when(i == NNB - 1)
    def _():
        p = pooled_ref[...]
        p = jnp.where(jnp.isfinite(p), p, 0.0)
        hh = jax.nn.relu(
            jnp.dot(p, w1_ref[...], preferred_element_type=jnp.float32, precision=lax.Precision.HIGHEST)
            + b1_ref[...][0][None, :])
        out_ref[...] = (
            jnp.dot(hh, w2_ref[...], preferred_element_type=jnp.float32, precision=lax.Precision.HIGHEST)
            + b2_ref[...][0][None, :])


@jax.jit
def _tc_final(agg, cnt16, bvec, batch, w1, b1, w2, b2):
    d = agg.shape[1]
    no = w2.shape[1]
    return pl.pallas_call(
        _final_body,
        grid=(NNB,),
        in_specs=[
            pl.BlockSpec((NB, d), lambda i: (i, 0)),
            pl.BlockSpec((NC, NB, 128), lambda i: (0, i, 0)),
            pl.BlockSpec((NB, 1), lambda i: (i, 0)),
            pl.BlockSpec((N,), lambda i: (0,), memory_space=pltpu.SMEM),
            pl.BlockSpec((d, d), lambda i: (0, 0)),
            pl.BlockSpec((1, d), lambda i: (0, 0)),
            pl.BlockSpec((d, no), lambda i: (0, 0)),
            pl.BlockSpec((1, no), lambda i: (0, 0)),
        ],
        out_specs=pl.BlockSpec((G, no), lambda i: (0, 0)),
        out_shape=jax.ShapeDtypeStruct((G, no), jnp.float32),
        scratch_shapes=[pltpu.VMEM((G, d), jnp.float32)],
    )(agg, cnt16, bvec, batch, w1, b1, w2, b2)


def _pad_to(a, rows, cols):
    return jnp.pad(a, ((0, rows - a.shape[0]), (0, cols - a.shape[1])))


def _conv(h_nodes, cnt16, dstc, srcc, layers, dg, first):
    dp = layers[0]["W"].shape[0] // 2
    dn = layers[0]["W"].shape[1]
    W1 = layers[0]["W"]
    A = _pad_to(W1[:dp] - W1[dp:], 128 if not first else dp, dg)
    B = _pad_to(W1[dp:], 128 if not first else dp, dg)
    if first:
        P, Q = _tc_project1(h_nodes, A, B)
    else:
        P, Q = _tc_project(h_nodes, cnt16, A, B)
    Y1, sp1 = _sc_gather_add(P, Q, dstc, srcc, d=dg)
    g = [_pad_to(lyr["gamma"].reshape(1, dn), 1, dg) for lyr in layers]
    b = [_pad_to(lyr["beta"].reshape(1, dn), 1, dg) for lyr in layers]
    W2 = _pad_to(layers[1]["W"], dg, dg)
    W3 = _pad_to(layers[2]["W"], dg, dg)
    Y2, s2 = _tc_mid(Y1, sp1, g[0], b[0], W2)
    Y3, s3 = _tc_mid(Y2, s2[None], g[1], b[1], W3)
    ss = _tc_scaleshift(s3[None], g[2], b[2], d=dg, ones_col=first)
    return _sc_scatter(Y3, ss, dstc, d=dg)


def kernel(x, edge_index, batch, params):
    srcc = edge_index[0].reshape(NCH, C)
    dstc = edge_index[1].reshape(NCH, C)
    agg1 = _conv(x, None, dstc, srcc, params["conv1"], 128, first=True)
    agg2 = _conv(agg1, agg1, dstc, srcc, params["conv2"], 128, first=False)
    agg = _conv(agg2, agg1, dstc, srcc, params["conv3"], 256, first=False)
    return _tc_final(
        agg, agg1, batch.reshape(N, 1), batch,
        params["fc1_W"], params["fc1_b"].reshape(1, -1),
        params["out_W"], params["out_b"].reshape(1, -1))
